# baseline (speedup 1.0000x reference)
import functools

import jax
import jax.numpy as jnp
from jax import lax
from jax.experimental import pallas as pl
from jax.experimental.pallas import tpu as pltpu
from jax.experimental.pallas import tpu_sc as plsc

_B = 65536
_NW = 32
_RPW = _B // _NW
_R = 128
_NCH = _RPW // _R
_NB = 3


def _sc_body(fe, a, c, o, obuf, *sems):
    in_s = sems[0:_NB]
    out_s = sems[_NB:2 * _NB]
    wid = lax.axis_index("s") * 2 + lax.axis_index("c")
    base = wid * _RPW

    def rows(k):
        return pl.ds(base + k * _R, _R)

    def in_copies(k):
        b = k % _NB
        return (
            pltpu.make_async_copy(fe.at[rows(k), :], obuf.at[b, :, 0:128], in_s[b]),
            pltpu.make_async_copy(c.at[rows(k), :], obuf.at[b, :, 128:192], in_s[b]),
            pltpu.make_async_copy(a.at[rows(k), :], obuf.at[b, :, 192:256], in_s[b]),
        )

    def out_copy(k):
        b = k % _NB
        return pltpu.make_async_copy(obuf.at[b], o.at[rows(k), :], out_s[b])

    for cp in in_copies(0):
        cp.start()
    for cp in in_copies(1):
        cp.start()
    for k in range(_NCH):
        for cp in in_copies(k):
            cp.wait()
        out_copy(k).start()
        if k + 2 < _NCH:
            if k >= 1:
                out_copy(k - 1).wait()
            for cp in in_copies(k + 2):
                cp.start()
    out_copy(_NCH - 2).wait()
    out_copy(_NCH - 1).wait()


def kernel(decoder_fe_output, decoder_alpha_output, decoder_carbon_output, idx_fe, idx_carbon, idx_alpha, out_dim):
    bsz = decoder_fe_output.shape[0]
    d_out = 256
    mesh = plsc.VectorSubcoreMesh(core_axis_name="c", subcore_axis_name="s")

    sck = functools.partial(
        pl.kernel,
        mesh=mesh,
        compiler_params=pltpu.CompilerParams(use_tc_tiling_on_sc=False),
        out_type=jax.ShapeDtypeStruct((bsz, d_out), jnp.float32),
        scratch_types=(
            [pltpu.VMEM((_NB, _R, d_out), jnp.float32)]
            + [pltpu.SemaphoreType.DMA] * (2 * _NB)
        ),
    )(_sc_body)
    return sck(decoder_fe_output, decoder_alpha_output, decoder_carbon_output)


# speedup vs baseline: 1.0118x; 1.0118x over previous
import functools

import jax
import jax.numpy as jnp
from jax import lax
from jax.experimental import pallas as pl
from jax.experimental.pallas import tpu as pltpu
from jax.experimental.pallas import tpu_sc as plsc

_B = 65536
_NSC = 2
_NPUMP = 4
_RPP = _B // (_NSC * _NPUMP)
_R = 512
_NCH = _RPP // _R
_NB = 4


def _sc_body(fe, a, c, o, sbuf, *sems):
    in_s = sems[0:_NB]
    out_s = sems[_NB:2 * _NB]
    cid = lax.axis_index("c")
    sid = lax.axis_index("s")
    base = cid * (_B // _NSC) + sid * _RPP

    def rows(k):
        return pl.ds(base + k * _R, _R)

    def in_copies(k):
        b = k % _NB
        return (
            pltpu.make_async_copy(fe.at[rows(k), :], sbuf.at[sid, b, :, 0:128], in_s[b]),
            pltpu.make_async_copy(c.at[rows(k), :], sbuf.at[sid, b, :, 128:192], in_s[b]),
            pltpu.make_async_copy(a.at[rows(k), :], sbuf.at[sid, b, :, 192:256], in_s[b]),
        )

    def out_copy(k):
        b = k % _NB
        return pltpu.make_async_copy(sbuf.at[sid, b], o.at[rows(k), :], out_s[b])

    @pl.when(sid < _NPUMP)
    def _():
        for j in range(_NB - 1):
            for cp in in_copies(j):
                cp.start()
        for k in range(_NCH):
            for cp in in_copies(k):
                cp.wait()
            out_copy(k).start()
            if k + (_NB - 1) < _NCH:
                if k >= 1:
                    out_copy(k - 1).wait()
                for cp in in_copies(k + (_NB - 1)):
                    cp.start()
        for k in range(_NCH - _NB, _NCH):
            out_copy(k).wait()


def kernel(decoder_fe_output, decoder_alpha_output, decoder_carbon_output, idx_fe, idx_carbon, idx_alpha, out_dim):
    bsz = decoder_fe_output.shape[0]
    d_out = 256
    mesh = plsc.VectorSubcoreMesh(core_axis_name="c", subcore_axis_name="s")

    sck = functools.partial(
        pl.kernel,
        mesh=mesh,
        compiler_params=pltpu.CompilerParams(use_tc_tiling_on_sc=False),
        out_type=jax.ShapeDtypeStruct((bsz, d_out), jnp.float32),
        scratch_types=(
            [pltpu.VMEM_SHARED((_NPUMP, _NB, _R, d_out), jnp.float32)]
            + [pltpu.SemaphoreType.DMA] * (2 * _NB)
        ),
    )(_sc_body)
    return sck(decoder_fe_output, decoder_alpha_output, decoder_carbon_output)
